# baseline (device time: 126999 ns/iter reference)
import jax
import jax.numpy as jnp
from jax import lax
from jax.experimental import pallas as pl
from jax.experimental.pallas import tpu as pltpu

N_DEV = 16


def kernel(A, B):
    m_per, k = A.shape
    _, n = B.shape

    def body(a_ref, b_ref, out_ref, gath_ref, send_sems, recv_sems):
        my_pos = lax.axis_index("i")
        left = (my_pos - 1) % N_DEV
        right = (my_pos + 1) % N_DEV

        barrier_sem = pltpu.get_barrier_semaphore()
        for nbr in (left, right):
            pl.semaphore_signal(
                barrier_sem, inc=1,
                device_id=(nbr,), device_id_type=pl.DeviceIdType.MESH,
            )
        pl.semaphore_wait(barrier_sem, 2)

        gath_ref[0, :, :] = a_ref[:, :]
        out_ref[pl.ds(my_pos * m_per, m_per), :] = jnp.dot(
            a_ref[:, :], b_ref[:, :], preferred_element_type=jnp.float32
        )

        for h in range(1, N_DEV):
            rdma = pltpu.make_async_remote_copy(
                src_ref=gath_ref.at[h - 1],
                dst_ref=gath_ref.at[h],
                send_sem=send_sems.at[h - 1],
                recv_sem=recv_sems.at[h - 1],
                device_id=(right,),
                device_id_type=pl.DeviceIdType.MESH,
            )
            rdma.start()
            rdma.wait()
            origin = (my_pos - h) % N_DEV
            out_ref[pl.ds(origin * m_per, m_per), :] = jnp.dot(
                gath_ref[h, :, :], b_ref[:, :],
                preferred_element_type=jnp.float32,
            )

    return pl.pallas_call(
        body,
        out_shape=jax.ShapeDtypeStruct((N_DEV * m_per, n), jnp.float32),
        in_specs=[
            pl.BlockSpec(memory_space=pltpu.VMEM),
            pl.BlockSpec(memory_space=pltpu.VMEM),
        ],
        out_specs=pl.BlockSpec(memory_space=pltpu.VMEM),
        scratch_shapes=[
            pltpu.VMEM((N_DEV, m_per, k), jnp.float32),
            pltpu.SemaphoreType.DMA((N_DEV - 1,)),
            pltpu.SemaphoreType.DMA((N_DEV - 1,)),
        ],
        compiler_params=pltpu.CompilerParams(collective_id=0),
    )(A, B)


# device time: 80593 ns/iter; 1.5758x vs baseline; 1.5758x over previous
import jax
import jax.numpy as jnp
from jax import lax
from jax.experimental import pallas as pl
from jax.experimental.pallas import tpu as pltpu

N_DEV = 16
CW_HOPS = N_DEV // 2
CCW_HOPS = N_DEV - 1 - CW_HOPS


def kernel(A, B):
    m_per, k = A.shape
    _, n = B.shape

    def body(a_ref, b_ref, out_ref, cw_ref, ccw_ref,
             cw_send, cw_recv, ccw_send, ccw_recv):
        my_pos = lax.axis_index("i")
        left = (my_pos - 1) % N_DEV
        right = (my_pos + 1) % N_DEV

        barrier_sem = pltpu.get_barrier_semaphore()
        for nbr in (left, right):
            pl.semaphore_signal(
                barrier_sem, inc=1,
                device_id=(nbr,), device_id_type=pl.DeviceIdType.MESH,
            )
        pl.semaphore_wait(barrier_sem, 2)

        cw_ref[0, :, :] = a_ref[:, :]
        ccw_ref[0, :, :] = a_ref[:, :]

        def gemm(src, origin):
            out_ref[pl.ds(origin * m_per, m_per), :] = jnp.dot(
                src, b_ref[:, :], preferred_element_type=jnp.float32
            )

        sends = []
        for h in range(1, CW_HOPS + 1):
            cw = pltpu.make_async_remote_copy(
                src_ref=cw_ref.at[h - 1],
                dst_ref=cw_ref.at[h],
                send_sem=cw_send.at[h - 1],
                recv_sem=cw_recv.at[h - 1],
                device_id=(right,),
                device_id_type=pl.DeviceIdType.MESH,
            )
            cw.start()
            sends.append(cw)
            if h <= CCW_HOPS:
                ccw = pltpu.make_async_remote_copy(
                    src_ref=ccw_ref.at[h - 1],
                    dst_ref=ccw_ref.at[h],
                    send_sem=ccw_send.at[h - 1],
                    recv_sem=ccw_recv.at[h - 1],
                    device_id=(left,),
                    device_id_type=pl.DeviceIdType.MESH,
                )
                ccw.start()
                sends.append(ccw)

            if h == 1:
                gemm(a_ref[:, :], my_pos)
            else:
                gemm(cw_ref[h - 1, :, :], (my_pos - (h - 1)) % N_DEV)
                if h - 1 <= CCW_HOPS:
                    gemm(ccw_ref[h - 1, :, :], (my_pos + (h - 1)) % N_DEV)

            cw.wait_recv()
            if h <= CCW_HOPS:
                ccw.wait_recv()

        gemm(cw_ref[CW_HOPS, :, :], (my_pos - CW_HOPS) % N_DEV)
        gemm(ccw_ref[CCW_HOPS, :, :], (my_pos + CCW_HOPS) % N_DEV)

        for s in sends:
            s.wait_send()

    return pl.pallas_call(
        body,
        out_shape=jax.ShapeDtypeStruct((N_DEV * m_per, n), jnp.float32),
        in_specs=[
            pl.BlockSpec(memory_space=pltpu.VMEM),
            pl.BlockSpec(memory_space=pltpu.VMEM),
        ],
        out_specs=pl.BlockSpec(memory_space=pltpu.VMEM),
        scratch_shapes=[
            pltpu.VMEM((CW_HOPS + 1, m_per, k), jnp.float32),
            pltpu.VMEM((CCW_HOPS + 1, m_per, k), jnp.float32),
            pltpu.SemaphoreType.DMA((CW_HOPS,)),
            pltpu.SemaphoreType.DMA((CW_HOPS,)),
            pltpu.SemaphoreType.DMA((CCW_HOPS,)),
            pltpu.SemaphoreType.DMA((CCW_HOPS,)),
        ],
        compiler_params=pltpu.CompilerParams(collective_id=0),
    )(A, B)


# device time: 63410 ns/iter; 2.0028x vs baseline; 1.2710x over previous
import jax
import jax.numpy as jnp
from jax import lax
from jax.experimental import pallas as pl
from jax.experimental.pallas import tpu as pltpu

N_DEV = 16
CW_HOPS = N_DEV // 2
CCW_HOPS = N_DEV - 1 - CW_HOPS


def kernel(A, B):
    m_per, k = A.shape
    _, n = B.shape
    half_m = m_per // 2

    def body(a_ref, b_ref, out_ref, cw_ref, ccw_ref,
             cw_send, cw_recv, ccw_send, ccw_recv):
        my_pos = lax.axis_index("i")
        left = (my_pos - 1) % N_DEV
        right = (my_pos + 1) % N_DEV

        barrier_sem = pltpu.get_barrier_semaphore()
        for nbr in (left, right):
            pl.semaphore_signal(
                barrier_sem, inc=1,
                device_id=(nbr,), device_id_type=pl.DeviceIdType.MESH,
            )
        pl.semaphore_wait(barrier_sem, 2)

        for half in range(2):
            rows = pl.ds(half * half_m, half_m)
            cw_ref[0, half, :, :] = a_ref[rows, :]
            ccw_ref[0, half, :, :] = a_ref[rows, :]

        sends = []

        def fwd(buf, send_sems, recv_sems, h, half, dst):
            r = pltpu.make_async_remote_copy(
                src_ref=buf.at[h - 1, half],
                dst_ref=buf.at[h, half],
                send_sem=send_sems.at[h - 1, half],
                recv_sem=recv_sems.at[h - 1, half],
                device_id=(dst,),
                device_id_type=pl.DeviceIdType.MESH,
            )
            r.start()
            sends.append(r)
            return r

        def gemm_half(src, origin, half):
            out_ref[pl.ds(origin * m_per + half * half_m, half_m), :] = jnp.dot(
                src, b_ref[:, :], preferred_element_type=jnp.float32
            )

        cw_a = fwd(cw_ref, cw_send, cw_recv, 1, 0, right)
        cw_b = fwd(cw_ref, cw_send, cw_recv, 1, 1, right)
        ccw_a = fwd(ccw_ref, ccw_send, ccw_recv, 1, 0, left)
        ccw_b = fwd(ccw_ref, ccw_send, ccw_recv, 1, 1, left)
        for half in range(2):
            gemm_half(a_ref[pl.ds(half * half_m, half_m), :], my_pos, half)

        for h in range(1, CW_HOPS + 1):
            cw_a.wait_recv()
            if h < CW_HOPS:
                cw_a = fwd(cw_ref, cw_send, cw_recv, h + 1, 0, right)
            cw_b.wait_recv()
            if h < CW_HOPS:
                cw_b = fwd(cw_ref, cw_send, cw_recv, h + 1, 1, right)
            if h <= CCW_HOPS:
                ccw_a.wait_recv()
                if h < CCW_HOPS:
                    ccw_a = fwd(ccw_ref, ccw_send, ccw_recv, h + 1, 0, left)
                ccw_b.wait_recv()
                if h < CCW_HOPS:
                    ccw_b = fwd(ccw_ref, ccw_send, ccw_recv, h + 1, 1, left)

            for half in range(2):
                gemm_half(cw_ref[h, half, :, :], (my_pos - h) % N_DEV, half)
            if h <= CCW_HOPS:
                for half in range(2):
                    gemm_half(ccw_ref[h, half, :, :], (my_pos + h) % N_DEV, half)

        for s in sends:
            s.wait_send()

    return pl.pallas_call(
        body,
        out_shape=jax.ShapeDtypeStruct((N_DEV * m_per, n), jnp.float32),
        in_specs=[
            pl.BlockSpec(memory_space=pltpu.VMEM),
            pl.BlockSpec(memory_space=pltpu.VMEM),
        ],
        out_specs=pl.BlockSpec(memory_space=pltpu.VMEM),
        scratch_shapes=[
            pltpu.VMEM((CW_HOPS + 1, 2, half_m, k), jnp.float32),
            pltpu.VMEM((CCW_HOPS + 1, 2, half_m, k), jnp.float32),
            pltpu.SemaphoreType.DMA((CW_HOPS, 2)),
            pltpu.SemaphoreType.DMA((CW_HOPS, 2)),
            pltpu.SemaphoreType.DMA((CCW_HOPS, 2)),
            pltpu.SemaphoreType.DMA((CCW_HOPS, 2)),
        ],
        compiler_params=pltpu.CompilerParams(collective_id=0),
    )(A, B)
